# baseline (speedup 1.0000x reference)
import jax
import jax.numpy as jnp
from jax import lax
from jax.experimental import pallas as pl
from jax.experimental.pallas import tpu as pltpu
from jax.experimental.pallas import tpu_sc as plsc

_B = 16384
_C = 5
_D = 64
_NW = 32
_BW = _B // _NW
_CH = 128
_NCH = _BW // _CH

_PARAMS = dict(
    compiler_params=pltpu.CompilerParams(use_tc_tiling_on_sc=False,
                                         needs_layout_passes=False),
)


def _k1_body(ctx_idx_hbm, ctx_table, wb_hbm, m_hbm,
             idx_a, ctx_a, idx_b, ctx_b, m_buf, wb_v, sem_a, sem_b):
    wid = lax.axis_index("s") * 2 + lax.axis_index("c")
    base = wid * _BW

    pltpu.sync_copy(wb_hbm, wb_v)
    bufs = [(idx_a, ctx_a, sem_a), (idx_b, ctx_b, sem_b)]

    def start_chunk(ch):
        idx_v, ctx_rows, sem = bufs[ch % 2]
        cb = base + ch * _CH
        pltpu.sync_copy(ctx_idx_hbm.at[cb // _CH], idx_v)
        cps = []
        for j in range(_C):
            cps.append(pltpu.async_copy(
                ctx_table.at[idx_v.at[j]],
                ctx_rows.at[pl.ds(j * _CH, _CH)], sem))
        return cps

    iota = lax.iota(jnp.int32, 16)
    kpats = [iota * 5 + 80 * q for q in range(4)]
    wsc = [plsc.load_gather(wb_v, [iota * 0 + j]) for j in range(_C)]

    pending = start_chunk(0)
    for ch in range(_NCH):
        nxt = start_chunk(ch + 1) if ch + 1 < _NCH else None
        for cp in pending:
            cp.wait()
        _, ctx_rows, _ = bufs[ch % 2]

        def body(e, carry):
            r0 = _C * e
            for q in range(4):
                acc = jnp.zeros((16,), jnp.float32)
                for j in range(_C):
                    k = kpats[q] + j
                    rowvec = lax.shift_right_logical(k, 6) + r0
                    colvec = lax.bitwise_and(k, 63)
                    cv = plsc.load_gather(ctx_rows, [rowvec, colvec])
                    acc = acc + cv * wsc[j]
                m_buf[e, pl.ds(16 * q, 16)] = acc
            return carry

        lax.fori_loop(0, _CH, body, 0)
        pltpu.sync_copy(m_buf, m_hbm.at[pl.ds(base + ch * _CH, _CH)])
        pending = nxt


def _k2_body(tgt_idx_hbm, tgt_table, m_hbm, wb_hbm, out_hbm,
             tidx_a, tgt_a, m_a, tidx_b, tgt_b, m_b, out_v, wb_v,
             sem_a, sem_b):
    wid = lax.axis_index("s") * 2 + lax.axis_index("c")
    base = wid * _BW

    pltpu.sync_copy(wb_hbm, wb_v)
    bufs = [(tidx_a, tgt_a, m_a, sem_a), (tidx_b, tgt_b, m_b, sem_b)]

    def start_chunk(ch):
        tidx_v, tgt_rows, m_rows, sem = bufs[ch % 2]
        cb = base + ch * _CH
        pltpu.sync_copy(tgt_idx_hbm.at[pl.ds(cb, _CH)], tidx_v)
        cps = [pltpu.async_copy(tgt_table.at[tidx_v], tgt_rows, sem),
               pltpu.async_copy(m_hbm.at[pl.ds(cb, _CH)], m_rows, sem)]
        return cps

    iota = lax.iota(jnp.int32, 16)
    bvec = plsc.load_gather(wb_v, [iota * 0 + _C])

    pending = start_chunk(0)
    for ch in range(_NCH):
        nxt = start_chunk(ch + 1) if ch + 1 < _NCH else None
        for cp in pending:
            cp.wait()
        _, tgt_rows, m_rows, _ = bufs[ch % 2]

        def gbody(g, carry):
            rowvec = iota + g * 16

            def dbody(dd, acc):
                colvec = iota * 0 + dd
                tv = plsc.load_gather(tgt_rows, [rowvec, colvec])
                mv = plsc.load_gather(m_rows, [rowvec, colvec])
                return acc + tv * mv

            acc = lax.fori_loop(0, _D, dbody, jnp.zeros((16,), jnp.float32))
            x = acc + bvec
            out_v[pl.ds(ch * _CH + g * 16, 16)] = 1.0 / (1.0 + jnp.exp(-x))
            return carry

        lax.fori_loop(0, _CH // 16, gbody, 0)
        pending = nxt

    pltpu.sync_copy(out_v, out_hbm.at[pl.ds(base, _BW)])


@jax.jit
def kernel(context_input, target_input, context_table, target_table,
           W_dense, b_dense):
    ctx_idx = context_input.reshape(_B // _CH, _C, _CH)
    tgt_idx = target_input.reshape(_B)
    wb = jnp.concatenate([W_dense.reshape(_C), b_dense,
                          jnp.zeros((2,), jnp.float32)])

    mesh = plsc.VectorSubcoreMesh(core_axis_name="c", subcore_axis_name="s",
                                  num_cores=2, num_subcores=16)
    k1 = pl.kernel(
        _k1_body,
        out_type=jax.ShapeDtypeStruct((_B, _D), jnp.float32),
        mesh=mesh,
        scratch_types=[
            pltpu.VMEM((_C, _CH), jnp.int32),
            pltpu.VMEM((_C * _CH, _D), jnp.float32),
            pltpu.VMEM((_C, _CH), jnp.int32),
            pltpu.VMEM((_C * _CH, _D), jnp.float32),
            pltpu.VMEM((_CH, _D), jnp.float32),
            pltpu.VMEM((8,), jnp.float32),
            pltpu.SemaphoreType.DMA,
            pltpu.SemaphoreType.DMA,
        ],
        **_PARAMS,
    )
    m = k1(ctx_idx, context_table, wb)

    k2 = pl.kernel(
        _k2_body,
        out_type=jax.ShapeDtypeStruct((_B,), jnp.float32),
        mesh=mesh,
        scratch_types=[
            pltpu.VMEM((_CH,), jnp.int32),
            pltpu.VMEM((_CH, _D), jnp.float32),
            pltpu.VMEM((_CH, _D), jnp.float32),
            pltpu.VMEM((_CH,), jnp.int32),
            pltpu.VMEM((_CH, _D), jnp.float32),
            pltpu.VMEM((_CH, _D), jnp.float32),
            pltpu.VMEM((_BW,), jnp.float32),
            pltpu.VMEM((8,), jnp.float32),
            pltpu.SemaphoreType.DMA,
            pltpu.SemaphoreType.DMA,
        ],
        **_PARAMS,
    )
    out = k2(tgt_idx, target_table, m, wb)
    return out.reshape(_B, 1)
